# Initial kernel scaffold; baseline (speedup 1.0000x reference)
import jax
import jax.numpy as jnp
from jax.experimental import pallas as pl


def kernel(x, edge_index, batch, W_enc, b_enc, W_convs, b_convs, W_ro, b_ro):
    raise NotImplementedError("write your pallas kernel here")


# speedup vs baseline: 13.0210x; 13.0210x over previous
import functools

import jax
import jax.numpy as jnp
from jax import lax
from jax.experimental import pallas as pl
from jax.experimental.pallas import tpu as pltpu
from jax.experimental.pallas import tpu_sc as plsc

N = 10000
E = 320000
D = 128
G = 64
NC = 2
NS = 16
CH = 128
E_SC = E // NC
E_T = E_SC // NS
NFULL = E_T // CH
TAIL = E_T - NFULL * CH
ROWS_T = 632

_MESH = plsc.VectorSubcoreMesh(core_axis_name="c", subcore_axis_name="s")


def _sc_degones_body(x_hbm, ones_hbm, dst_hbm, out_hbm,
                     acc_sh, ones_v, dst_v, tdst_v):
    c = lax.axis_index("c")
    s = lax.axis_index("s")
    r0 = jnp.minimum(s * ROWS_T, N - ROWS_T)
    pltpu.sync_copy(x_hbm.at[pl.ds(r0, ROWS_T)], acc_sh.at[pl.ds(r0, ROWS_T)])
    pltpu.sync_copy(ones_hbm, ones_v)
    plsc.subcore_barrier()
    ebase = c * E_SC + s * E_T

    def chunk(i, carry):
        pltpu.sync_copy(dst_hbm.at[pl.ds(ebase + i * CH, CH)], dst_v)
        pltpu.sync_copy(ones_v, acc_sh.at[dst_v], add=True)
        return carry

    lax.fori_loop(0, NFULL, chunk, 0)
    pltpu.sync_copy(dst_hbm.at[pl.ds(ebase + NFULL * CH, TAIL)], tdst_v)
    pltpu.sync_copy(ones_v.at[pl.ds(0, TAIL)], acc_sh.at[tdst_v], add=True)
    plsc.subcore_barrier()
    pltpu.sync_copy(acc_sh.at[pl.ds(r0, ROWS_T)], out_hbm.at[c, pl.ds(r0, ROWS_T)])


_sc_degones = functools.partial(
    pl.kernel,
    out_type=jax.ShapeDtypeStruct((NC, N, D), jnp.float32),
    mesh=_MESH,
    scratch_types=[
        pltpu.VMEM_SHARED((N, D), jnp.float32),
        pltpu.VMEM((CH, D), jnp.float32),
        pltpu.VMEM((CH,), jnp.int32),
        pltpu.VMEM((TAIL,), jnp.int32),
    ],
)(_sc_degones_body)


def _sc_scatter_body(m_hbm, src_hbm, dst_hbm, out_hbm,
                     acc_sh, src_v, dst_v, rows_v, tsrc_v, tdst_v, trows_v, sem):
    c = lax.axis_index("c")
    s = lax.axis_index("s")
    r0 = jnp.minimum(s * ROWS_T, N - ROWS_T)
    pltpu.sync_copy(m_hbm.at[pl.ds(r0, ROWS_T)], acc_sh.at[pl.ds(r0, ROWS_T)])
    plsc.subcore_barrier()
    ebase = c * E_SC + s * E_T

    def chunk(i, carry):
        off = ebase + i * CH
        pltpu.sync_copy(src_hbm.at[pl.ds(off, CH)], src_v)
        pltpu.sync_copy(dst_hbm.at[pl.ds(off, CH)], dst_v)
        pltpu.async_copy(m_hbm.at[src_v], rows_v, sem).wait()
        pltpu.sync_copy(rows_v, acc_sh.at[dst_v], add=True)
        return carry

    lax.fori_loop(0, NFULL, chunk, 0)
    off = ebase + NFULL * CH
    pltpu.sync_copy(src_hbm.at[pl.ds(off, TAIL)], tsrc_v)
    pltpu.sync_copy(dst_hbm.at[pl.ds(off, TAIL)], tdst_v)
    pltpu.async_copy(m_hbm.at[tsrc_v], trows_v, sem).wait()
    pltpu.sync_copy(trows_v, acc_sh.at[tdst_v], add=True)
    plsc.subcore_barrier()
    pltpu.sync_copy(acc_sh.at[pl.ds(r0, ROWS_T)], out_hbm.at[c, pl.ds(r0, ROWS_T)])


_sc_scatter = functools.partial(
    pl.kernel,
    out_type=jax.ShapeDtypeStruct((NC, N, D), jnp.float32),
    mesh=_MESH,
    scratch_types=[
        pltpu.VMEM_SHARED((N, D), jnp.float32),
        pltpu.VMEM((CH,), jnp.int32),
        pltpu.VMEM((CH,), jnp.int32),
        pltpu.VMEM((CH, D), jnp.float32),
        pltpu.VMEM((TAIL,), jnp.int32),
        pltpu.VMEM((TAIL,), jnp.int32),
        pltpu.VMEM((TAIL, D), jnp.float32),
        pltpu.SemaphoreType.DMA,
    ],
)(_sc_scatter_body)


RB = 2000
GRID = N // RB


def _tc_enc_body(degp_ref, x_ref, we_ref, be_ref, w0_ref, m0_ref, dinv_ref):
    deg = degp_ref[0, :, 0] + degp_ref[1, :, 0] - 2.0 * x_ref[:, 0] + 1.0
    dinv = lax.rsqrt(deg)[:, None]
    h = jnp.dot(x_ref[...], we_ref[...], preferred_element_type=jnp.float32)
    h = h + be_ref[...]
    m0_ref[...] = dinv * jnp.dot(h, w0_ref[...], preferred_element_type=jnp.float32)
    dinv_ref[...] = dinv


def _tc_enc(degp, x, W_enc, b_enc, W0):
    return pl.pallas_call(
        _tc_enc_body,
        grid=(GRID,),
        in_specs=[
            pl.BlockSpec((NC, RB, D), lambda i: (0, i, 0)),
            pl.BlockSpec((RB, D), lambda i: (i, 0)),
            pl.BlockSpec((D, D), lambda i: (0, 0)),
            pl.BlockSpec((1, D), lambda i: (0, 0)),
            pl.BlockSpec((D, D), lambda i: (0, 0)),
        ],
        out_specs=[
            pl.BlockSpec((RB, D), lambda i: (i, 0)),
            pl.BlockSpec((RB, 1), lambda i: (i, 0)),
        ],
        out_shape=[
            jax.ShapeDtypeStruct((N, D), jnp.float32),
            jax.ShapeDtypeStruct((N, 1), jnp.float32),
        ],
    )(degp, x, W_enc, b_enc.reshape(1, D), W0)


def _tc_mid_body(accp_ref, mprev_ref, dinv_ref, b_ref, wn_ref, mnext_ref):
    acc = accp_ref[0] + accp_ref[1] - mprev_ref[...]
    pre = dinv_ref[...] * acc + b_ref[...]
    h = pre * jax.nn.sigmoid(pre)
    mnext_ref[...] = dinv_ref[...] * jnp.dot(
        h, wn_ref[...], preferred_element_type=jnp.float32)


def _tc_mid(accp, m_prev, dinv, b, W_next):
    return pl.pallas_call(
        _tc_mid_body,
        grid=(GRID,),
        in_specs=[
            pl.BlockSpec((NC, RB, D), lambda i: (0, i, 0)),
            pl.BlockSpec((RB, D), lambda i: (i, 0)),
            pl.BlockSpec((RB, 1), lambda i: (i, 0)),
            pl.BlockSpec((1, D), lambda i: (0, 0)),
            pl.BlockSpec((D, D), lambda i: (0, 0)),
        ],
        out_specs=pl.BlockSpec((RB, D), lambda i: (i, 0)),
        out_shape=jax.ShapeDtypeStruct((N, D), jnp.float32),
    )(accp, m_prev, dinv, b.reshape(1, D), W_next)


def _tc_final_body(accp_ref, mprev_ref, dinv_ref, b_ref, batch_ref, wro_ref,
                   bro_ref, out_ref, pooled_scr):
    i = pl.program_id(0)

    @pl.when(i == 0)
    def _():
        pooled_scr[...] = jnp.zeros_like(pooled_scr)

    acc = accp_ref[0] + accp_ref[1] - mprev_ref[...]
    pre = dinv_ref[...] * acc + b_ref[...]
    h = pre * jax.nn.sigmoid(pre)
    onehot = (batch_ref[...] ==
              lax.broadcasted_iota(jnp.int32, (RB, G), 1)).astype(jnp.float32)
    pooled_scr[...] += lax.dot_general(
        onehot, h, (((0,), (0,)), ((), ())), preferred_element_type=jnp.float32)

    @pl.when(i == GRID - 1)
    def _():
        out = jnp.dot(pooled_scr[...], wro_ref[...],
                      preferred_element_type=jnp.float32) + bro_ref[...]
        out_ref[...] = jnp.maximum(out, 0.0)


def _tc_final(accp, m_prev, dinv, b, batch2d, W_ro, b_ro):
    C = W_ro.shape[1]
    return pl.pallas_call(
        _tc_final_body,
        grid=(GRID,),
        in_specs=[
            pl.BlockSpec((NC, RB, D), lambda i: (0, i, 0)),
            pl.BlockSpec((RB, D), lambda i: (i, 0)),
            pl.BlockSpec((RB, 1), lambda i: (i, 0)),
            pl.BlockSpec((1, D), lambda i: (0, 0)),
            pl.BlockSpec((RB, 1), lambda i: (i, 0)),
            pl.BlockSpec((D, C), lambda i: (0, 0)),
            pl.BlockSpec((1, C), lambda i: (0, 0)),
        ],
        out_specs=pl.BlockSpec((G, C), lambda i: (0, 0)),
        out_shape=jax.ShapeDtypeStruct((G, C), jnp.float32),
        scratch_shapes=[pltpu.VMEM((G, D), jnp.float32)],
    )(accp, m_prev, dinv, b.reshape(1, D), batch2d, W_ro, b_ro.reshape(1, C))


def kernel(x, edge_index, batch, W_enc, b_enc, W_convs, b_convs, W_ro, b_ro):
    src = edge_index[0]
    dst = edge_index[1]
    ones = jnp.ones((CH, D), jnp.float32)
    degp = _sc_degones(x, ones, dst)
    m, dinv = _tc_enc(degp, x, W_enc, b_enc, W_convs[0])
    for i in range(W_convs.shape[0]):
        accp = _sc_scatter(m, src, dst)
        if i < W_convs.shape[0] - 1:
            m = _tc_mid(accp, m, dinv, b_convs[i], W_convs[i + 1])
    return _tc_final(accp, m, dinv, b_convs[-1], batch.reshape(N, 1), W_ro, b_ro)


# speedup vs baseline: 24.9065x; 1.9128x over previous
import functools

import jax
import jax.numpy as jnp
from jax import lax
from jax.experimental import pallas as pl
from jax.experimental.pallas import tpu as pltpu
from jax.experimental.pallas import tpu_sc as plsc

N = 10000
E = 320000
D = 128
G = 64
NC = 2
NS = 16
CH = 128
NR_T = 80
ER = NC * NS * NR_T
PAD = ER * CH - E
NPAIR = NR_T // 2 - 1
ROWS_T = 632

_MESH = plsc.VectorSubcoreMesh(core_axis_name="c", subcore_axis_name="s")


def _sc_degones_body(x_hbm, ones_hbm, dst_hbm, out_hbm,
                     acc_sh, ones_v, dst_loc, ssem):
    c = lax.axis_index("c")
    s = lax.axis_index("s")
    w = c * NS + s
    r0 = jnp.minimum(s * ROWS_T, N - ROWS_T)
    pltpu.sync_copy(x_hbm.at[pl.ds(r0, ROWS_T)], acc_sh.at[pl.ds(r0, ROWS_T)])
    pltpu.sync_copy(ones_hbm, ones_v)
    pltpu.sync_copy(dst_hbm.at[pl.ds(w * NR_T, NR_T)], dst_loc)
    plsc.subcore_barrier()

    def fire(j, carry):
        pltpu.async_copy(ones_v, acc_sh.at[dst_loc.at[j]], ssem, add=True)
        return carry

    lax.fori_loop(0, NR_T, fire, 0)

    def drain(j, carry):
        pltpu.make_async_copy(ones_hbm, ones_v, ssem).wait()
        return carry

    lax.fori_loop(0, NR_T, drain, 0)
    plsc.subcore_barrier()
    pltpu.sync_copy(acc_sh.at[pl.ds(r0, ROWS_T)], out_hbm.at[c, pl.ds(r0, ROWS_T)])


_sc_degones = functools.partial(
    pl.kernel,
    out_type=jax.ShapeDtypeStruct((NC, N, D), jnp.float32),
    mesh=_MESH,
    scratch_types=[
        pltpu.VMEM_SHARED((N, D), jnp.float32),
        pltpu.VMEM((CH, D), jnp.float32),
        pltpu.VMEM((NR_T, CH), jnp.int32),
        pltpu.SemaphoreType.DMA,
    ],
)(_sc_degones_body)


IDXROWS = 48


def _jj(j):
    return jnp.where(j >= IDXROWS, j - IDXROWS, j)


def _sc_scatter_body(m_hbm, src_hbm, dst_hbm, out_hbm,
                     acc_sh, src_loc, dst_loc, rows0, rows1, gsem0, gsem1):
    c = lax.axis_index("c")
    s = lax.axis_index("s")
    w = c * NS + s
    r0 = jnp.minimum(s * ROWS_T, N - ROWS_T)
    pltpu.sync_copy(m_hbm.at[pl.ds(r0, ROWS_T)], acc_sh.at[pl.ds(r0, ROWS_T)])
    eb = w * NR_T
    pltpu.sync_copy(src_hbm.at[pl.ds(eb, IDXROWS)], src_loc)
    pltpu.sync_copy(dst_hbm.at[pl.ds(eb, IDXROWS)], dst_loc)
    plsc.subcore_barrier()

    pltpu.async_copy(m_hbm.at[src_loc.at[0]], rows0, gsem0)
    pltpu.async_copy(m_hbm.at[src_loc.at[1]], rows1, gsem1)

    def pair(p, carry):
        j = 2 * p
        pltpu.make_async_copy(m_hbm.at[src_loc.at[_jj(j)]], rows0, gsem0).wait()
        pltpu.sync_copy(rows0, acc_sh.at[dst_loc.at[_jj(j)]], add=True)

        @pl.when(p == 21)
        def _():
            pltpu.sync_copy(src_hbm.at[pl.ds(eb + IDXROWS, NR_T - IDXROWS)],
                            src_loc.at[pl.ds(0, NR_T - IDXROWS)])
            pltpu.sync_copy(dst_hbm.at[pl.ds(eb + IDXROWS, NR_T - IDXROWS)],
                            dst_loc.at[pl.ds(0, NR_T - IDXROWS)])

        pltpu.async_copy(m_hbm.at[src_loc.at[_jj(j + 2)]], rows0, gsem0)
        pltpu.make_async_copy(m_hbm.at[src_loc.at[_jj(j + 1)]], rows1, gsem1).wait()
        pltpu.sync_copy(rows1, acc_sh.at[dst_loc.at[_jj(j + 1)]], add=True)
        pltpu.async_copy(m_hbm.at[src_loc.at[_jj(j + 3)]], rows1, gsem1)
        return carry

    lax.fori_loop(0, NPAIR, pair, 0)
    j = 2 * NPAIR
    pltpu.make_async_copy(m_hbm.at[src_loc.at[_jj(j)]], rows0, gsem0).wait()
    pltpu.sync_copy(rows0, acc_sh.at[dst_loc.at[_jj(j)]], add=True)
    pltpu.make_async_copy(m_hbm.at[src_loc.at[_jj(j + 1)]], rows1, gsem1).wait()
    pltpu.sync_copy(rows1, acc_sh.at[dst_loc.at[_jj(j + 1)]], add=True)
    plsc.subcore_barrier()
    pltpu.sync_copy(acc_sh.at[pl.ds(r0, ROWS_T)], out_hbm.at[c, pl.ds(r0, ROWS_T)])


_sc_scatter = functools.partial(
    pl.kernel,
    out_type=jax.ShapeDtypeStruct((NC, N, D), jnp.float32),
    mesh=_MESH,
    scratch_types=[
        pltpu.VMEM_SHARED((N, D), jnp.float32),
        pltpu.VMEM((IDXROWS, CH), jnp.int32),
        pltpu.VMEM((IDXROWS, CH), jnp.int32),
        pltpu.VMEM((CH, D), jnp.float32),
        pltpu.VMEM((CH, D), jnp.float32),
        pltpu.SemaphoreType.DMA,
        pltpu.SemaphoreType.DMA,
    ],
)(_sc_scatter_body)


RB = 2000
GRID = N // RB


def _pad_mask(i):
    rows = lax.broadcasted_iota(jnp.int32, (RB, 1), 0) + i * RB
    return (rows < PAD).astype(jnp.float32)


def _tc_enc_body(degp_ref, x_ref, we_ref, be_ref, w0_ref, m0_ref, dinv_ref):
    i = pl.program_id(0)
    deg = (degp_ref[0, :, 0] + degp_ref[1, :, 0] - 2.0 * x_ref[:, 0] + 1.0
           - _pad_mask(i)[:, 0])
    dinv = lax.rsqrt(deg)[:, None]
    h = jnp.dot(x_ref[...], we_ref[...], preferred_element_type=jnp.float32)
    h = h + be_ref[...]
    m0_ref[...] = dinv * jnp.dot(h, w0_ref[...], preferred_element_type=jnp.float32)
    dinv_ref[...] = dinv


def _tc_enc(degp, x, W_enc, b_enc, W0):
    return pl.pallas_call(
        _tc_enc_body,
        grid=(GRID,),
        in_specs=[
            pl.BlockSpec((NC, RB, D), lambda i: (0, i, 0)),
            pl.BlockSpec((RB, D), lambda i: (i, 0)),
            pl.BlockSpec((D, D), lambda i: (0, 0)),
            pl.BlockSpec((1, D), lambda i: (0, 0)),
            pl.BlockSpec((D, D), lambda i: (0, 0)),
        ],
        out_specs=[
            pl.BlockSpec((RB, D), lambda i: (i, 0)),
            pl.BlockSpec((RB, 1), lambda i: (i, 0)),
        ],
        out_shape=[
            jax.ShapeDtypeStruct((N, D), jnp.float32),
            jax.ShapeDtypeStruct((N, 1), jnp.float32),
        ],
    )(degp, x, W_enc, b_enc.reshape(1, D), W0)


def _tc_mid_body(accp_ref, mprev_ref, dinv_ref, b_ref, wn_ref, mnext_ref):
    i = pl.program_id(0)
    acc = (accp_ref[0] + accp_ref[1]
           - (1.0 + _pad_mask(i)) * mprev_ref[...])
    pre = dinv_ref[...] * acc + b_ref[...]
    h = pre * jax.nn.sigmoid(pre)
    mnext_ref[...] = dinv_ref[...] * jnp.dot(
        h, wn_ref[...], preferred_element_type=jnp.float32)


def _tc_mid(accp, m_prev, dinv, b, W_next):
    return pl.pallas_call(
        _tc_mid_body,
        grid=(GRID,),
        in_specs=[
            pl.BlockSpec((NC, RB, D), lambda i: (0, i, 0)),
            pl.BlockSpec((RB, D), lambda i: (i, 0)),
            pl.BlockSpec((RB, 1), lambda i: (i, 0)),
            pl.BlockSpec((1, D), lambda i: (0, 0)),
            pl.BlockSpec((D, D), lambda i: (0, 0)),
        ],
        out_specs=pl.BlockSpec((RB, D), lambda i: (i, 0)),
        out_shape=jax.ShapeDtypeStruct((N, D), jnp.float32),
    )(accp, m_prev, dinv, b.reshape(1, D), W_next)


def _tc_final_body(accp_ref, mprev_ref, dinv_ref, b_ref, batch_ref, wro_ref,
                   bro_ref, out_ref, pooled_scr):
    i = pl.program_id(0)

    @pl.when(i == 0)
    def _():
        pooled_scr[...] = jnp.zeros_like(pooled_scr)

    acc = (accp_ref[0] + accp_ref[1]
           - (1.0 + _pad_mask(i)) * mprev_ref[...])
    pre = dinv_ref[...] * acc + b_ref[...]
    h = pre * jax.nn.sigmoid(pre)
    onehot = (batch_ref[...] ==
              lax.broadcasted_iota(jnp.int32, (RB, G), 1)).astype(jnp.float32)
    pooled_scr[...] += lax.dot_general(
        onehot, h, (((0,), (0,)), ((), ())), preferred_element_type=jnp.float32)

    @pl.when(i == GRID - 1)
    def _():
        out = jnp.dot(pooled_scr[...], wro_ref[...],
                      preferred_element_type=jnp.float32) + bro_ref[...]
        out_ref[...] = jnp.maximum(out, 0.0)


def _tc_final(accp, m_prev, dinv, b, batch2d, W_ro, b_ro):
    C = W_ro.shape[1]
    return pl.pallas_call(
        _tc_final_body,
        grid=(GRID,),
        in_specs=[
            pl.BlockSpec((NC, RB, D), lambda i: (0, i, 0)),
            pl.BlockSpec((RB, D), lambda i: (i, 0)),
            pl.BlockSpec((RB, 1), lambda i: (i, 0)),
            pl.BlockSpec((1, D), lambda i: (0, 0)),
            pl.BlockSpec((RB, 1), lambda i: (i, 0)),
            pl.BlockSpec((D, C), lambda i: (0, 0)),
            pl.BlockSpec((1, C), lambda i: (0, 0)),
        ],
        out_specs=pl.BlockSpec((G, C), lambda i: (0, 0)),
        out_shape=jax.ShapeDtypeStruct((G, C), jnp.float32),
        scratch_shapes=[pltpu.VMEM((G, D), jnp.float32)],
    )(accp, m_prev, dinv, b.reshape(1, D), batch2d, W_ro, b_ro.reshape(1, C))


def kernel(x, edge_index, batch, W_enc, b_enc, W_convs, b_convs, W_ro, b_ro):
    padk = jnp.arange(PAD, dtype=jnp.int32)
    src = jnp.concatenate([edge_index[0], padk]).reshape(ER, CH)
    dst = jnp.concatenate([edge_index[1], padk]).reshape(ER, CH)
    ones = jnp.ones((CH, D), jnp.float32)
    degp = _sc_degones(x, ones, dst)
    m, dinv = _tc_enc(degp, x, W_enc, b_enc, W_convs[0])
    for i in range(W_convs.shape[0]):
        accp = _sc_scatter(m, src, dst)
        if i < W_convs.shape[0] - 1:
            m = _tc_mid(accp, m, dinv, b_convs[i], W_convs[i + 1])
    return _tc_final(accp, m, dinv, b_convs[-1], batch.reshape(N, 1), W_ro, b_ro)


# speedup vs baseline: 25.1941x; 1.0116x over previous
import functools

import jax
import jax.numpy as jnp
from jax import lax
from jax.experimental import pallas as pl
from jax.experimental.pallas import tpu as pltpu
from jax.experimental.pallas import tpu_sc as plsc

N = 10000
E = 320000
D = 128
G = 64
NC = 2
NS = 16
CH = 128
NR_T = 80
ER = NC * NS * NR_T
PAD = ER * CH - E
NPAIR = NR_T // 2 - 1
ROWS_T = 632

_MESH = plsc.VectorSubcoreMesh(core_axis_name="c", subcore_axis_name="s")


def _sc_degones_body(x_hbm, ones_hbm, dst_hbm, out_hbm,
                     acc_sh, ones_v, dst_loc, ssem):
    c = lax.axis_index("c")
    s = lax.axis_index("s")
    w = c * NS + s
    r0 = jnp.minimum(s * ROWS_T, N - ROWS_T)
    pltpu.async_copy(x_hbm.at[pl.ds(r0, ROWS_T)], acc_sh.at[pl.ds(r0, ROWS_T)],
                     ssem)
    pltpu.sync_copy(ones_hbm, ones_v)
    pltpu.sync_copy(dst_hbm.at[pl.ds(w * NR_T, NR_T)], dst_loc)
    pltpu.make_async_copy(x_hbm.at[pl.ds(r0, ROWS_T)],
                          acc_sh.at[pl.ds(r0, ROWS_T)], ssem).wait()
    plsc.subcore_barrier()

    def fire(j, carry):
        pltpu.async_copy(ones_v, acc_sh.at[dst_loc.at[j]], ssem, add=True)
        return carry

    lax.fori_loop(0, NR_T, fire, 0)

    def drain(j, carry):
        pltpu.make_async_copy(ones_hbm, ones_v, ssem).wait()
        return carry

    lax.fori_loop(0, NR_T, drain, 0)
    plsc.subcore_barrier()
    pltpu.sync_copy(acc_sh.at[pl.ds(r0, ROWS_T)], out_hbm.at[c, pl.ds(r0, ROWS_T)])


_sc_degones = functools.partial(
    pl.kernel,
    out_type=jax.ShapeDtypeStruct((NC, N, D), jnp.float32),
    mesh=_MESH,
    scratch_types=[
        pltpu.VMEM_SHARED((N, D), jnp.float32),
        pltpu.VMEM((CH, D), jnp.float32),
        pltpu.VMEM((NR_T, CH), jnp.int32),
        pltpu.SemaphoreType.DMA,
    ],
)(_sc_degones_body)


IDXROWS = 48


def _jj(j):
    return jnp.where(j >= IDXROWS, j - IDXROWS, j)


def _sc_scatter_body(m_hbm, src_hbm, dst_hbm, out_hbm,
                     acc_sh, src_loc, dst_loc, rows0, rows1, gsem0, gsem1):
    c = lax.axis_index("c")
    s = lax.axis_index("s")
    w = c * NS + s
    r0 = jnp.minimum(s * ROWS_T, N - ROWS_T)
    pltpu.async_copy(m_hbm.at[pl.ds(r0, ROWS_T)], acc_sh.at[pl.ds(r0, ROWS_T)],
                     gsem0)
    eb = w * NR_T
    pltpu.sync_copy(src_hbm.at[pl.ds(eb, IDXROWS)], src_loc)
    pltpu.sync_copy(dst_hbm.at[pl.ds(eb, IDXROWS)], dst_loc)
    pltpu.make_async_copy(m_hbm.at[pl.ds(r0, ROWS_T)],
                          acc_sh.at[pl.ds(r0, ROWS_T)], gsem0).wait()
    plsc.subcore_barrier()

    pltpu.async_copy(m_hbm.at[src_loc.at[0]], rows0, gsem0)
    pltpu.async_copy(m_hbm.at[src_loc.at[1]], rows1, gsem1)

    def pair(p, carry):
        j = 2 * p
        pltpu.make_async_copy(m_hbm.at[src_loc.at[_jj(j)]], rows0, gsem0).wait()
        pltpu.sync_copy(rows0, acc_sh.at[dst_loc.at[_jj(j)]], add=True)

        @pl.when(p == 21)
        def _():
            pltpu.sync_copy(src_hbm.at[pl.ds(eb + IDXROWS, NR_T - IDXROWS)],
                            src_loc.at[pl.ds(0, NR_T - IDXROWS)])
            pltpu.sync_copy(dst_hbm.at[pl.ds(eb + IDXROWS, NR_T - IDXROWS)],
                            dst_loc.at[pl.ds(0, NR_T - IDXROWS)])

        pltpu.async_copy(m_hbm.at[src_loc.at[_jj(j + 2)]], rows0, gsem0)
        pltpu.make_async_copy(m_hbm.at[src_loc.at[_jj(j + 1)]], rows1, gsem1).wait()
        pltpu.sync_copy(rows1, acc_sh.at[dst_loc.at[_jj(j + 1)]], add=True)
        pltpu.async_copy(m_hbm.at[src_loc.at[_jj(j + 3)]], rows1, gsem1)
        return carry

    lax.fori_loop(0, NPAIR, pair, 0)
    j = 2 * NPAIR
    pltpu.make_async_copy(m_hbm.at[src_loc.at[_jj(j)]], rows0, gsem0).wait()
    pltpu.sync_copy(rows0, acc_sh.at[dst_loc.at[_jj(j)]], add=True)
    pltpu.make_async_copy(m_hbm.at[src_loc.at[_jj(j + 1)]], rows1, gsem1).wait()
    pltpu.sync_copy(rows1, acc_sh.at[dst_loc.at[_jj(j + 1)]], add=True)
    plsc.subcore_barrier()
    pltpu.sync_copy(acc_sh.at[pl.ds(r0, ROWS_T)], out_hbm.at[c, pl.ds(r0, ROWS_T)])


_sc_scatter = functools.partial(
    pl.kernel,
    out_type=jax.ShapeDtypeStruct((NC, N, D), jnp.float32),
    mesh=_MESH,
    scratch_types=[
        pltpu.VMEM_SHARED((N, D), jnp.float32),
        pltpu.VMEM((IDXROWS, CH), jnp.int32),
        pltpu.VMEM((IDXROWS, CH), jnp.int32),
        pltpu.VMEM((CH, D), jnp.float32),
        pltpu.VMEM((CH, D), jnp.float32),
        pltpu.SemaphoreType.DMA,
        pltpu.SemaphoreType.DMA,
    ],
)(_sc_scatter_body)


RB = 2000
GRID = N // RB


def _pad_mask(i):
    rows = lax.broadcasted_iota(jnp.int32, (RB, 1), 0) + i * RB
    return (rows < PAD).astype(jnp.float32)


def _tc_enc_body(degp_ref, x_ref, we_ref, be_ref, w0_ref, m0_ref, dinv_ref):
    i = pl.program_id(0)
    deg = (degp_ref[0, :, 0] + degp_ref[1, :, 0] - 2.0 * x_ref[:, 0] + 1.0
           - _pad_mask(i)[:, 0])
    dinv = lax.rsqrt(deg)[:, None]
    h = jnp.dot(x_ref[...], we_ref[...], preferred_element_type=jnp.float32)
    h = h + be_ref[...]
    m0_ref[...] = dinv * jnp.dot(h, w0_ref[...], preferred_element_type=jnp.float32)
    dinv_ref[...] = dinv


def _tc_enc(degp, x, W_enc, b_enc, W0):
    return pl.pallas_call(
        _tc_enc_body,
        grid=(GRID,),
        in_specs=[
            pl.BlockSpec((NC, RB, D), lambda i: (0, i, 0)),
            pl.BlockSpec((RB, D), lambda i: (i, 0)),
            pl.BlockSpec((D, D), lambda i: (0, 0)),
            pl.BlockSpec((1, D), lambda i: (0, 0)),
            pl.BlockSpec((D, D), lambda i: (0, 0)),
        ],
        out_specs=[
            pl.BlockSpec((RB, D), lambda i: (i, 0)),
            pl.BlockSpec((RB, 1), lambda i: (i, 0)),
        ],
        out_shape=[
            jax.ShapeDtypeStruct((N, D), jnp.float32),
            jax.ShapeDtypeStruct((N, 1), jnp.float32),
        ],
    )(degp, x, W_enc, b_enc.reshape(1, D), W0)


def _tc_mid_body(accp_ref, mprev_ref, dinv_ref, b_ref, wn_ref, mnext_ref):
    i = pl.program_id(0)
    acc = (accp_ref[0] + accp_ref[1]
           - (1.0 + _pad_mask(i)) * mprev_ref[...])
    pre = dinv_ref[...] * acc + b_ref[...]
    h = pre * jax.nn.sigmoid(pre)
    mnext_ref[...] = dinv_ref[...] * jnp.dot(
        h, wn_ref[...], preferred_element_type=jnp.float32)


def _tc_mid(accp, m_prev, dinv, b, W_next):
    return pl.pallas_call(
        _tc_mid_body,
        grid=(GRID,),
        in_specs=[
            pl.BlockSpec((NC, RB, D), lambda i: (0, i, 0)),
            pl.BlockSpec((RB, D), lambda i: (i, 0)),
            pl.BlockSpec((RB, 1), lambda i: (i, 0)),
            pl.BlockSpec((1, D), lambda i: (0, 0)),
            pl.BlockSpec((D, D), lambda i: (0, 0)),
        ],
        out_specs=pl.BlockSpec((RB, D), lambda i: (i, 0)),
        out_shape=jax.ShapeDtypeStruct((N, D), jnp.float32),
    )(accp, m_prev, dinv, b.reshape(1, D), W_next)


def _tc_final_body(accp_ref, mprev_ref, dinv_ref, b_ref, batch_ref, wro_ref,
                   bro_ref, out_ref, pooled_scr):
    i = pl.program_id(0)

    @pl.when(i == 0)
    def _():
        pooled_scr[...] = jnp.zeros_like(pooled_scr)

    acc = (accp_ref[0] + accp_ref[1]
           - (1.0 + _pad_mask(i)) * mprev_ref[...])
    pre = dinv_ref[...] * acc + b_ref[...]
    h = pre * jax.nn.sigmoid(pre)
    onehot = (batch_ref[...] ==
              lax.broadcasted_iota(jnp.int32, (RB, G), 1)).astype(jnp.float32)
    pooled_scr[...] += lax.dot_general(
        onehot, h, (((0,), (0,)), ((), ())), preferred_element_type=jnp.float32)

    @pl.when(i == GRID - 1)
    def _():
        out = jnp.dot(pooled_scr[...], wro_ref[...],
                      preferred_element_type=jnp.float32) + bro_ref[...]
        out_ref[...] = jnp.maximum(out, 0.0)


def _tc_final(accp, m_prev, dinv, b, batch2d, W_ro, b_ro):
    C = W_ro.shape[1]
    return pl.pallas_call(
        _tc_final_body,
        grid=(GRID,),
        in_specs=[
            pl.BlockSpec((NC, RB, D), lambda i: (0, i, 0)),
            pl.BlockSpec((RB, D), lambda i: (i, 0)),
            pl.BlockSpec((RB, 1), lambda i: (i, 0)),
            pl.BlockSpec((1, D), lambda i: (0, 0)),
            pl.BlockSpec((RB, 1), lambda i: (i, 0)),
            pl.BlockSpec((D, C), lambda i: (0, 0)),
            pl.BlockSpec((1, C), lambda i: (0, 0)),
        ],
        out_specs=pl.BlockSpec((G, C), lambda i: (0, 0)),
        out_shape=jax.ShapeDtypeStruct((G, C), jnp.float32),
        scratch_shapes=[pltpu.VMEM((G, D), jnp.float32)],
    )(accp, m_prev, dinv, b.reshape(1, D), batch2d, W_ro, b_ro.reshape(1, C))


def kernel(x, edge_index, batch, W_enc, b_enc, W_convs, b_convs, W_ro, b_ro):
    padk = jnp.arange(PAD, dtype=jnp.int32)
    src = jnp.concatenate([edge_index[0], padk]).reshape(ER, CH)
    dst = jnp.concatenate([edge_index[1], padk]).reshape(ER, CH)
    ones = jnp.ones((CH, D), jnp.float32)
    degp = _sc_degones(x, ones, dst)
    m, dinv = _tc_enc(degp, x, W_enc, b_enc, W_convs[0])
    for i in range(W_convs.shape[0]):
        accp = _sc_scatter(m, src, dst)
        if i < W_convs.shape[0] - 1:
            m = _tc_mid(accp, m, dinv, b_convs[i], W_convs[i + 1])
    return _tc_final(accp, m, dinv, b_convs[-1], batch.reshape(N, 1), W_ro, b_ro)
